# baseline (speedup 1.0000x reference)
import jax
import jax.numpy as jnp
from jax import lax
from jax.experimental import pallas as pl
from jax.experimental.pallas import tpu as pltpu
from jax.experimental.pallas import tpu_sc as plsc

L_E = 8192
D = 1152
NLANE = 16
NTILE = 32
ROWS_PER_TILE = L_E // NTILE
CHUNK = 16
NCHUNK = ROWS_PER_TILE // CHUNK
NBUF = 4
RBLK = 8
KCH = D // NLANE
COPY_BLOCK = 512


def _vrsqrt(x):
    i = plsc.bitcast(x, jnp.int32)
    i = jnp.int32(0x5F3759DF) - lax.shift_right_logical(i, 1)
    y = plsc.bitcast(i, jnp.float32)
    half = jnp.float32(0.5)
    three_half = jnp.float32(1.5)
    for _ in range(3):
        y = y * (three_half - half * x * y * y)
    return y


def _sc_scan_body(mt_hbm, mem_hbm, csim_hbm, cidx_hbm,
                  mt_v, b0, b1, b2, b3, dscr, sscr, csim_v, cidx_v,
                  si0, si1, si2, si3):
    bufs = (b0, b1, b2, b3)
    isems = (si0, si1, si2, si3)
    c = lax.axis_index("c")
    s = lax.axis_index("s")
    wid = c * 16 + s
    base_row = wid * ROWS_PER_TILE
    pltpu.sync_copy(mt_hbm, mt_v)
    lane = lax.iota(jnp.int32, NLANE)

    def in_cp(ch, b):
        return pltpu.make_async_copy(
            mem_hbm.at[pl.ds(base_row + ch * CHUNK, CHUNK)], bufs[b],
            isems[b])

    for b in range(NBUF - 1):
        in_cp(b, b).start()

    def compute(buf, row0, best_sim, best_idx):
        dvecs, svecs = [], []
        for half in range(NLANE // RBLK):
            r0 = half * RBLK

            def kbody(k, accs):
                dacc, sacc = accs
                mtk = mt_v[pl.ds(k * NLANE, NLANE)]
                nd, ns = [], []
                for r in range(RBLK):
                    v = buf[r0 + r, pl.ds(k * NLANE, NLANE)]
                    nd.append(dacc[r] + v * mtk)
                    ns.append(sacc[r] + v * v)
                return tuple(nd), tuple(ns)

            zeros = tuple(jnp.zeros((NLANE,), jnp.float32)
                          for _ in range(RBLK))
            d8, s8 = lax.fori_loop(0, KCH, kbody, (zeros, zeros), unroll=2)
            dvecs.extend(d8)
            svecs.extend(s8)
        for r in range(NLANE):
            dscr[r, pl.ds(0, NLANE)] = dvecs[r]
            sscr[r, pl.ds(0, NLANE)] = svecs[r]
        dsum = jnp.zeros((NLANE,), jnp.float32)
        ssum = jnp.zeros((NLANE,), jnp.float32)
        for col in range(NLANE):
            cvec = jnp.full((NLANE,), col, jnp.int32)
            dsum = dsum + plsc.load_gather(dscr, [lane, cvec])
            ssum = ssum + plsc.load_gather(sscr, [lane, cvec])
        simv = dsum * _vrsqrt(ssum)
        rows = row0 + lane
        take = simv > best_sim
        best_sim = jnp.where(take, simv, best_sim)
        best_idx = jnp.where(take, rows, best_idx)
        return best_sim, best_idx

    def quad(q, best):
        best_sim, best_idx = best
        for j in range(NBUF):
            ch = NBUF * q + j
            in_cp(ch, j).wait()

            @pl.when(ch + NBUF - 1 < NCHUNK)
            def _():
                in_cp(ch + NBUF - 1, (j + NBUF - 1) % NBUF).start()

            best_sim, best_idx = compute(
                bufs[j], base_row + ch * CHUNK, best_sim, best_idx)
        return best_sim, best_idx

    init = (jnp.full((NLANE,), -jnp.inf, jnp.float32),
            jnp.full((NLANE,), 2**30, jnp.int32))
    best_sim, best_idx = lax.fori_loop(0, NCHUNK // NBUF, quad, init)

    csim_v[...] = best_sim
    cidx_v[...] = best_idx
    pltpu.sync_copy(csim_v, csim_hbm.at[wid])
    pltpu.sync_copy(cidx_v, cidx_hbm.at[wid])


_sc_scan = pl.kernel(
    _sc_scan_body,
    out_type=(
        jax.ShapeDtypeStruct((NTILE, NLANE), jnp.float32),
        jax.ShapeDtypeStruct((NTILE, NLANE), jnp.int32),
    ),
    mesh=plsc.VectorSubcoreMesh(core_axis_name="c", subcore_axis_name="s"),
    compiler_params=pltpu.CompilerParams(needs_layout_passes=False),
    scratch_types=[
        pltpu.VMEM((D,), jnp.float32),
        pltpu.VMEM((CHUNK, D), jnp.float32),
        pltpu.VMEM((CHUNK, D), jnp.float32),
        pltpu.VMEM((CHUNK, D), jnp.float32),
        pltpu.VMEM((CHUNK, D), jnp.float32),
        pltpu.VMEM((NLANE, NLANE + 1), jnp.float32),
        pltpu.VMEM((NLANE, NLANE + 1), jnp.float32),
        pltpu.VMEM((NLANE,), jnp.float32),
        pltpu.VMEM((NLANE,), jnp.int32),
        pltpu.SemaphoreType.DMA,
        pltpu.SemaphoreType.DMA,
        pltpu.SemaphoreType.DMA,
        pltpu.SemaphoreType.DMA,
    ],
)


def _copy_body(src_ref, dst_ref):
    dst_ref[...] = src_ref[...]


def _tc_copy(memory):
    return pl.pallas_call(
        _copy_body,
        grid=(L_E // COPY_BLOCK,),
        in_specs=[pl.BlockSpec((COPY_BLOCK, D), lambda i: (i, 0))],
        out_specs=pl.BlockSpec((COPY_BLOCK, D), lambda i: (i, 0)),
        out_shape=jax.ShapeDtypeStruct((L_E, D), jnp.float32),
    )(memory)


def _fixup_body(sim_ref, idx_ref, mt_ref, src_ref, out_ref, sem):
    del src_ref
    sims = sim_ref[...]
    idxs = idx_ref[...]
    m = jnp.max(sims)
    winner = jnp.min(jnp.where(sims == m, idxs, jnp.int32(2**30)))
    cp = pltpu.make_async_copy(mt_ref, out_ref.at[pl.ds(winner, 1)], sem)
    cp.start()
    cp.wait()


def kernel(M_t, memory):
    csim, cidx = _sc_scan(M_t, memory)
    copied = _tc_copy(memory)
    out = pl.pallas_call(
        _fixup_body,
        out_shape=jax.ShapeDtypeStruct((L_E, D), jnp.float32),
        in_specs=[
            pl.BlockSpec(memory_space=pltpu.VMEM),
            pl.BlockSpec(memory_space=pltpu.VMEM),
            pl.BlockSpec(memory_space=pltpu.VMEM),
            pl.BlockSpec(memory_space=pl.ANY),
        ],
        out_specs=pl.BlockSpec(memory_space=pl.ANY),
        scratch_shapes=[pltpu.SemaphoreType.DMA],
        input_output_aliases={3: 0},
    )(csim.reshape(4, 128), cidx.reshape(4, 128), M_t.reshape(1, D), copied)
    return out


# speedup vs baseline: 1.1160x; 1.1160x over previous
import jax
import jax.numpy as jnp
from jax import lax
from jax.experimental import pallas as pl
from jax.experimental.pallas import tpu as pltpu
from jax.experimental.pallas import tpu_sc as plsc

L_E = 8192
D = 1152
NLANE = 16
NTILE = 32
ROWS_PER_TILE = L_E // NTILE
CHUNK = 16
NCHUNK = ROWS_PER_TILE // CHUNK
NBUF = 4
RBLK = 8
KCH = D // NLANE


def _rsqrt32(x):
    i = lax.bitcast_convert_type(x, jnp.int32)
    i = jnp.int32(0x5F3759DF) - lax.shift_right_logical(i, 1)
    y = lax.bitcast_convert_type(i, jnp.float32)
    half = jnp.float32(0.5)
    three_half = jnp.float32(1.5)
    for _ in range(3):
        y = y * (three_half - half * x * y * y)
    return y


def _sc_scan_body(mt_hbm, mem_hbm, out_hbm, csim_hbm, cidx_hbm,
                  mt_v, b0, b1, b2, b3, csim_v, cidx_v,
                  si0, si1, si2, si3, so0, so1, so2, so3):
    bufs = (b0, b1, b2, b3)
    isems = (si0, si1, si2, si3)
    osems = (so0, so1, so2, so3)
    c = lax.axis_index("c")
    s = lax.axis_index("s")
    wid = c * 16 + s
    base_row = wid * ROWS_PER_TILE
    pltpu.sync_copy(mt_hbm, mt_v)

    def in_cp(ch, b):
        return pltpu.make_async_copy(
            mem_hbm.at[pl.ds(base_row + ch * CHUNK, CHUNK)], bufs[b],
            isems[b])

    def out_cp(ch, b):
        return pltpu.make_async_copy(
            bufs[b], out_hbm.at[pl.ds(base_row + ch * CHUNK, CHUNK)],
            osems[b])

    in_cp(0, 0).start()
    in_cp(1, 1).start()

    def compute(buf, row0, best_sim, best_idx):
        for half in range(CHUNK // RBLK):
            r0 = half * RBLK

            def kbody(k, accs):
                dacc, sacc = accs
                mtk = mt_v[pl.ds(k * NLANE, NLANE)]
                nd, ns = [], []
                for r in range(RBLK):
                    v = buf[r0 + r, pl.ds(k * NLANE, NLANE)]
                    nd.append(dacc[r] + v * mtk)
                    ns.append(sacc[r] + v * v)
                return tuple(nd), tuple(ns)

            zeros = tuple(jnp.zeros((NLANE,), jnp.float32)
                          for _ in range(RBLK))
            dvecs, svecs = lax.fori_loop(0, KCH, kbody, (zeros, zeros),
                                         unroll=4)
            for r in range(RBLK):
                dsum = jnp.sum(dvecs[r])
                ssum = jnp.sum(svecs[r])
                sim = dsum * _rsqrt32(ssum)
                ridx = row0 + r0 + r
                take = sim > best_sim
                best_sim = jnp.where(take, sim, best_sim)
                best_idx = jnp.where(take, ridx, best_idx)
        return best_sim, best_idx

    def quad(q, best):
        best_sim, best_idx = best
        for j in range(NBUF):
            ch = NBUF * q + j
            in_cp(ch, j).wait()
            best_sim, best_idx = compute(
                bufs[j], base_row + ch * CHUNK, best_sim, best_idx)
            out_cp(ch, j).start()
            nxt = ch + 2
            bb = (j + 2) % NBUF

            @pl.when(nxt < NCHUNK)
            def _():
                @pl.when(ch >= 2)
                def _():
                    out_cp(ch - 2, bb).wait()
                in_cp(nxt, bb).start()
        return best_sim, best_idx

    init = (jnp.float32(-jnp.inf), jnp.int32(0))
    best_sim, best_idx = lax.fori_loop(0, NCHUNK // NBUF, quad, init)

    for b in range(NBUF):
        out_cp(NCHUNK - NBUF + b, b).wait()

    lane = lax.iota(jnp.int32, NLANE)
    csim_v[...] = jnp.where(lane == 0, best_sim, jnp.float32(-jnp.inf))
    cidx_v[...] = jnp.where(lane == 0, best_idx, jnp.int32(2**30))
    pltpu.sync_copy(csim_v, csim_hbm.at[wid])
    pltpu.sync_copy(cidx_v, cidx_hbm.at[wid])


_sc_scan = pl.kernel(
    _sc_scan_body,
    out_type=(
        jax.ShapeDtypeStruct((L_E, D), jnp.float32),
        jax.ShapeDtypeStruct((NTILE, NLANE), jnp.float32),
        jax.ShapeDtypeStruct((NTILE, NLANE), jnp.int32),
    ),
    mesh=plsc.VectorSubcoreMesh(core_axis_name="c", subcore_axis_name="s"),
    compiler_params=pltpu.CompilerParams(needs_layout_passes=False),
    scratch_types=[
        pltpu.VMEM((D,), jnp.float32),
        pltpu.VMEM((CHUNK, D), jnp.float32),
        pltpu.VMEM((CHUNK, D), jnp.float32),
        pltpu.VMEM((CHUNK, D), jnp.float32),
        pltpu.VMEM((CHUNK, D), jnp.float32),
        pltpu.VMEM((NLANE,), jnp.float32),
        pltpu.VMEM((NLANE,), jnp.int32),
        pltpu.SemaphoreType.DMA,
        pltpu.SemaphoreType.DMA,
        pltpu.SemaphoreType.DMA,
        pltpu.SemaphoreType.DMA,
        pltpu.SemaphoreType.DMA,
        pltpu.SemaphoreType.DMA,
        pltpu.SemaphoreType.DMA,
        pltpu.SemaphoreType.DMA,
    ],
)


def _fixup_body(sim_ref, idx_ref, mt_ref, src_ref, out_ref, sem):
    del src_ref
    sims = sim_ref[...]
    idxs = idx_ref[...]
    m = jnp.max(sims)
    winner = jnp.min(jnp.where(sims == m, idxs, jnp.int32(2**30)))
    cp = pltpu.make_async_copy(mt_ref, out_ref.at[pl.ds(winner, 1)], sem)
    cp.start()
    cp.wait()


def kernel(M_t, memory):
    copied, csim, cidx = _sc_scan(M_t, memory)
    out = pl.pallas_call(
        _fixup_body,
        out_shape=jax.ShapeDtypeStruct((L_E, D), jnp.float32),
        in_specs=[
            pl.BlockSpec(memory_space=pltpu.VMEM),
            pl.BlockSpec(memory_space=pltpu.VMEM),
            pl.BlockSpec(memory_space=pltpu.VMEM),
            pl.BlockSpec(memory_space=pl.ANY),
        ],
        out_specs=pl.BlockSpec(memory_space=pl.ANY),
        scratch_shapes=[pltpu.SemaphoreType.DMA],
        input_output_aliases={3: 0},
    )(csim.reshape(4, 128), cidx.reshape(4, 128), M_t.reshape(1, D), copied)
    return out


# speedup vs baseline: 1.1494x; 1.0299x over previous
import jax
import jax.numpy as jnp
from jax import lax
from jax.experimental import pallas as pl
from jax.experimental.pallas import tpu as pltpu
from jax.experimental.pallas import tpu_sc as plsc

L_E = 8192
D = 1152
NLANE = 16
NTILE = 32
ROWS_PER_TILE = L_E // NTILE
CHUNK = 16
NCHUNK = ROWS_PER_TILE // CHUNK
NBUF = 4
RBLK = 16
KCH = D // NLANE


def _rsqrt32(x):
    i = lax.bitcast_convert_type(x, jnp.int32)
    i = jnp.int32(0x5F3759DF) - lax.shift_right_logical(i, 1)
    y = lax.bitcast_convert_type(i, jnp.float32)
    half = jnp.float32(0.5)
    three_half = jnp.float32(1.5)
    for _ in range(3):
        y = y * (three_half - half * x * y * y)
    return y


def _sc_scan_body(mt_hbm, mem_hbm, out_hbm, csim_hbm, cidx_hbm,
                  mt_v, b0, b1, b2, b3, csim_v, cidx_v,
                  si0, si1, si2, si3, so0, so1, so2, so3):
    bufs = (b0, b1, b2, b3)
    isems = (si0, si1, si2, si3)
    osems = (so0, so1, so2, so3)
    c = lax.axis_index("c")
    s = lax.axis_index("s")
    wid = c * 16 + s
    base_row = wid * ROWS_PER_TILE
    pltpu.sync_copy(mt_hbm, mt_v)

    def in_cp(ch, b):
        return pltpu.make_async_copy(
            mem_hbm.at[pl.ds(base_row + ch * CHUNK, CHUNK)], bufs[b],
            isems[b])

    def out_cp(ch, b):
        return pltpu.make_async_copy(
            bufs[b], out_hbm.at[pl.ds(base_row + ch * CHUNK, CHUNK)],
            osems[b])

    in_cp(0, 0).start()
    in_cp(1, 1).start()

    def compute(buf, row0, best_sim, best_idx):
        for half in range(CHUNK // RBLK):
            r0 = half * RBLK

            def kbody(k, accs):
                dacc, sacc = accs
                mtk = mt_v[pl.ds(k * NLANE, NLANE)]
                nd, ns = [], []
                for r in range(RBLK):
                    v = buf[r0 + r, pl.ds(k * NLANE, NLANE)]
                    nd.append(dacc[r] + v * mtk)
                    ns.append(sacc[r] + v * v)
                return tuple(nd), tuple(ns)

            zeros = tuple(jnp.zeros((NLANE,), jnp.float32)
                          for _ in range(RBLK))
            dvecs, svecs = lax.fori_loop(0, KCH, kbody, (zeros, zeros))
            for r in range(RBLK):
                dsum = jnp.sum(dvecs[r])
                ssum = jnp.sum(svecs[r])
                sim = dsum * _rsqrt32(ssum)
                ridx = row0 + r0 + r
                take = sim > best_sim
                best_sim = jnp.where(take, sim, best_sim)
                best_idx = jnp.where(take, ridx, best_idx)
        return best_sim, best_idx

    def quad(q, best):
        best_sim, best_idx = best
        for j in range(NBUF):
            ch = NBUF * q + j
            in_cp(ch, j).wait()
            best_sim, best_idx = compute(
                bufs[j], base_row + ch * CHUNK, best_sim, best_idx)
            out_cp(ch, j).start()
            nxt = ch + 2
            bb = (j + 2) % NBUF

            @pl.when(nxt < NCHUNK)
            def _():
                @pl.when(ch >= 2)
                def _():
                    out_cp(ch - 2, bb).wait()
                in_cp(nxt, bb).start()
        return best_sim, best_idx

    init = (jnp.float32(-jnp.inf), jnp.int32(0))
    best_sim, best_idx = lax.fori_loop(0, NCHUNK // NBUF, quad, init)

    for b in range(NBUF):
        out_cp(NCHUNK - NBUF + b, b).wait()

    lane = lax.iota(jnp.int32, NLANE)
    csim_v[...] = jnp.where(lane == 0, best_sim, jnp.float32(-jnp.inf))
    cidx_v[...] = jnp.where(lane == 0, best_idx, jnp.int32(2**30))
    pltpu.sync_copy(csim_v, csim_hbm.at[wid])
    pltpu.sync_copy(cidx_v, cidx_hbm.at[wid])


_sc_scan = pl.kernel(
    _sc_scan_body,
    out_type=(
        jax.ShapeDtypeStruct((L_E, D), jnp.float32),
        jax.ShapeDtypeStruct((NTILE, NLANE), jnp.float32),
        jax.ShapeDtypeStruct((NTILE, NLANE), jnp.int32),
    ),
    mesh=plsc.VectorSubcoreMesh(core_axis_name="c", subcore_axis_name="s"),
    compiler_params=pltpu.CompilerParams(needs_layout_passes=False),
    scratch_types=[
        pltpu.VMEM((D,), jnp.float32),
        pltpu.VMEM((CHUNK, D), jnp.float32),
        pltpu.VMEM((CHUNK, D), jnp.float32),
        pltpu.VMEM((CHUNK, D), jnp.float32),
        pltpu.VMEM((CHUNK, D), jnp.float32),
        pltpu.VMEM((NLANE,), jnp.float32),
        pltpu.VMEM((NLANE,), jnp.int32),
        pltpu.SemaphoreType.DMA,
        pltpu.SemaphoreType.DMA,
        pltpu.SemaphoreType.DMA,
        pltpu.SemaphoreType.DMA,
        pltpu.SemaphoreType.DMA,
        pltpu.SemaphoreType.DMA,
        pltpu.SemaphoreType.DMA,
        pltpu.SemaphoreType.DMA,
    ],
)


def _fixup_body(sim_ref, idx_ref, mt_ref, src_ref, out_ref, sem):
    del src_ref
    sims = sim_ref[...]
    idxs = idx_ref[...]
    m = jnp.max(sims)
    winner = jnp.min(jnp.where(sims == m, idxs, jnp.int32(2**30)))
    cp = pltpu.make_async_copy(mt_ref, out_ref.at[pl.ds(winner, 1)], sem)
    cp.start()
    cp.wait()


def kernel(M_t, memory):
    copied, csim, cidx = _sc_scan(M_t, memory)
    out = pl.pallas_call(
        _fixup_body,
        out_shape=jax.ShapeDtypeStruct((L_E, D), jnp.float32),
        in_specs=[
            pl.BlockSpec(memory_space=pltpu.VMEM),
            pl.BlockSpec(memory_space=pltpu.VMEM),
            pl.BlockSpec(memory_space=pltpu.VMEM),
            pl.BlockSpec(memory_space=pl.ANY),
        ],
        out_specs=pl.BlockSpec(memory_space=pl.ANY),
        scratch_shapes=[pltpu.SemaphoreType.DMA],
        input_output_aliases={3: 0},
    )(csim.reshape(4, 128), cidx.reshape(4, 128), M_t.reshape(1, D), copied)
    return out
